# baseline (speedup 1.0000x reference)
import functools

import jax
import jax.numpy as jnp
from jax import lax
from jax.experimental import pallas as pl
from jax.experimental.pallas import tpu as pltpu
from jax.experimental.pallas import tpu_sc as plsc

VOCAB = 100000
NEMBED = 128
BATCH = 4096
CTX = 20

_HB = BATCH // 2


_NC = 2
_NS = 16
_NW = _NC * _NS
_BPW = _HB // _NW
_CHUNK = 4
_NCHUNK = _BPW // _CHUNK
_LANES = NEMBED // 16

_sc_mesh = plsc.VectorSubcoreMesh(core_axis_name="c", subcore_axis_name="s")


@functools.partial(
    pl.kernel,
    mesh=_sc_mesh,
    out_type=jax.ShapeDtypeStruct((_HB, NEMBED), jnp.float32),
    scratch_types=[
        pltpu.VMEM((_BPW * CTX,), jnp.int32),
        pltpu.VMEM((_CHUNK * CTX, NEMBED), jnp.float32),
        pltpu.VMEM((_CHUNK * CTX, NEMBED), jnp.float32),
        pltpu.VMEM((_BPW, NEMBED), jnp.float32),
        pltpu.SemaphoreType.DMA,
        pltpu.SemaphoreType.DMA,
    ],
)
def _gather_sum(idx_hbm, table_hbm, out_hbm, idx_v, rows_a, rows_b, acc_v,
                sem_a, sem_b):
    wid = lax.axis_index("s") * _NC + lax.axis_index("c")
    base = wid * _BPW
    n_idx = _CHUNK * CTX
    pltpu.sync_copy(idx_hbm.at[pl.ds(base * CTX, _BPW * CTX)], idx_v)

    def _start(ci, buf, sem):
        pltpu.async_copy(table_hbm.at[idx_v.at[pl.ds(ci * n_idx, n_idx)]],
                         buf, sem)

    def _wait(buf, sem):
        pltpu.make_async_copy(table_hbm.at[pl.ds(0, n_idx)], buf, sem).wait()

    def _accum(ci, buf):
        for r in range(_CHUNK):
            accs = [buf[r * CTX, pl.ds(d * 16, 16)] for d in range(_LANES)]
            for c in range(1, CTX):
                for d in range(_LANES):
                    accs[d] = accs[d] + buf[r * CTX + c, pl.ds(d * 16, 16)]
            for d in range(_LANES):
                acc_v[ci * _CHUNK + r, pl.ds(d * 16, 16)] = accs[d]

    _start(0, rows_a, sem_a)
    _start(1, rows_b, sem_b)

    def pair_body(j, carry):
        c0 = j * 2
        _wait(rows_a, sem_a)
        _accum(c0, rows_a)

        @pl.when(c0 + 2 < _NCHUNK)
        def _():
            _start(c0 + 2, rows_a, sem_a)

        _wait(rows_b, sem_b)
        _accum(c0 + 1, rows_b)

        @pl.when(c0 + 3 < _NCHUNK)
        def _():
            _start(c0 + 3, rows_b, sem_b)

        return carry

    lax.fori_loop(0, _NCHUNK // 2, pair_body, 0)
    pltpu.sync_copy(acc_v, out_hbm.at[pl.ds(base, _BPW)])


_VB = 2048


def _make_matmul(col_block, aliased):
    def body(*refs):
        if aliased:
            _, w_ref, x_ref, b_ref, o_ref = refs
        else:
            w_ref, x_ref, b_ref, o_ref = refs
        w = w_ref[...].astype(jnp.bfloat16)
        x = (x_ref[...] * (1.0 / CTX)).astype(jnp.bfloat16)
        acc = lax.dot_general(
            w, x, (((1,), (1,)), ((), ())), preferred_element_type=jnp.float32
        )
        o_ref[...] = acc + b_ref[...]

    in_specs = [
        pl.BlockSpec((_VB, NEMBED), lambda v: (v, 0)),
        pl.BlockSpec((_HB, NEMBED), lambda v: (0, 0)),
        pl.BlockSpec((_VB, 1), lambda v: (v, 0)),
    ]
    kwargs = {}
    if aliased:
        in_specs = [pl.BlockSpec(memory_space=pl.ANY)] + in_specs
        kwargs["input_output_aliases"] = {0: 0}
    return pl.pallas_call(
        body,
        grid=(pl.cdiv(VOCAB, _VB),),
        in_specs=in_specs,
        out_specs=pl.BlockSpec((_VB, _HB), lambda v: (v, col_block)),
        out_shape=jax.ShapeDtypeStruct((VOCAB, BATCH), jnp.float32),
        compiler_params=pltpu.CompilerParams(
            dimension_semantics=("arbitrary",),
        ),
        **kwargs,
    )


_matmul_half0 = _make_matmul(0, aliased=False)
_matmul_half1 = _make_matmul(1, aliased=True)


def kernel(inp, embed_table, W, b):
    idx = inp.reshape(-1).astype(jnp.int32)
    avg0 = _gather_sum(idx[: _HB * CTX], embed_table)
    avg1 = _gather_sum(idx[_HB * CTX:], embed_table)
    b2 = b.reshape(VOCAB, 1)
    out = _matmul_half0(W, avg0, b2)
    out = _matmul_half1(out, W, avg1, b2)
    return out.T


# speedup vs baseline: 1.0631x; 1.0631x over previous
import functools

import jax
import jax.numpy as jnp
from jax import lax
from jax.experimental import pallas as pl
from jax.experimental.pallas import tpu as pltpu
from jax.experimental.pallas import tpu_sc as plsc

VOCAB = 100000
NEMBED = 128
BATCH = 4096
CTX = 20


_NC = 2
_NS = 16
_NW = _NC * _NS
_BPW = BATCH // _NW
_CHUNK = 4
_NCHUNK = _BPW // _CHUNK
_LANES = NEMBED // 16

_sc_mesh = plsc.VectorSubcoreMesh(core_axis_name="c", subcore_axis_name="s")


@functools.partial(
    pl.kernel,
    mesh=_sc_mesh,
    out_type=jax.ShapeDtypeStruct((BATCH, NEMBED), jnp.float32),
    scratch_types=[
        pltpu.VMEM((_BPW * CTX,), jnp.int32),
        pltpu.VMEM((_CHUNK * CTX, NEMBED), jnp.float32),
        pltpu.VMEM((_CHUNK * CTX, NEMBED), jnp.float32),
        pltpu.VMEM((_BPW, NEMBED), jnp.float32),
        pltpu.SemaphoreType.DMA,
        pltpu.SemaphoreType.DMA,
    ],
)
def _gather_sum(idx_hbm, table_hbm, out_hbm, idx_v, rows_a, rows_b, acc_v,
                sem_a, sem_b):
    wid = lax.axis_index("s") * _NC + lax.axis_index("c")
    base = wid * _BPW
    n_idx = _CHUNK * CTX
    pltpu.sync_copy(idx_hbm.at[pl.ds(base * CTX, _BPW * CTX)], idx_v)

    def _start(ci, buf, sem):
        pltpu.async_copy(table_hbm.at[idx_v.at[pl.ds(ci * n_idx, n_idx)]],
                         buf, sem)

    def _wait(buf, sem):
        pltpu.make_async_copy(table_hbm.at[pl.ds(0, n_idx)], buf, sem).wait()

    def _accum(ci, buf):
        for r in range(_CHUNK):
            accs = [buf[r * CTX, pl.ds(d * 16, 16)] for d in range(_LANES)]
            for c in range(1, CTX):
                for d in range(_LANES):
                    accs[d] = accs[d] + buf[r * CTX + c, pl.ds(d * 16, 16)]
            for d in range(_LANES):
                acc_v[ci * _CHUNK + r, pl.ds(d * 16, 16)] = accs[d]

    _start(0, rows_a, sem_a)
    _start(1, rows_b, sem_b)

    def pair_body(j, carry):
        c0 = j * 2
        _wait(rows_a, sem_a)
        _accum(c0, rows_a)

        @pl.when(c0 + 2 < _NCHUNK)
        def _():
            _start(c0 + 2, rows_a, sem_a)

        _wait(rows_b, sem_b)
        _accum(c0 + 1, rows_b)

        @pl.when(c0 + 3 < _NCHUNK)
        def _():
            _start(c0 + 3, rows_b, sem_b)

        return carry

    lax.fori_loop(0, _NCHUNK // 2, pair_body, 0)
    pltpu.sync_copy(acc_v, out_hbm.at[pl.ds(base, _BPW)])


_VB = 1024


def _mm_body(w_ref, x_ref, b_ref, o_ref):
    w = w_ref[...].astype(jnp.bfloat16)
    x = (x_ref[...] * (1.0 / CTX)).astype(jnp.bfloat16)
    acc = lax.dot_general(
        w, x, (((1,), (1,)), ((), ())), preferred_element_type=jnp.float32
    )
    o_ref[...] = acc + b_ref[...]


_matmul_t = pl.pallas_call(
    _mm_body,
    grid=(pl.cdiv(VOCAB, _VB),),
    in_specs=[
        pl.BlockSpec((_VB, NEMBED), lambda v: (v, 0)),
        pl.BlockSpec((BATCH, NEMBED), lambda v: (0, 0)),
        pl.BlockSpec((_VB, 1), lambda v: (v, 0)),
    ],
    out_specs=pl.BlockSpec((_VB, BATCH), lambda v: (v, 0)),
    out_shape=jax.ShapeDtypeStruct((VOCAB, BATCH), jnp.float32),
    compiler_params=pltpu.CompilerParams(
        dimension_semantics=("arbitrary",),
    ),
)


def kernel(inp, embed_table, W, b):
    idx = inp.reshape(-1).astype(jnp.int32)
    avg = _gather_sum(idx, embed_table)
    return _matmul_t(W, avg, b.reshape(VOCAB, 1)).T


# speedup vs baseline: 1.0671x; 1.0038x over previous
import functools

import jax
import jax.numpy as jnp
from jax import lax
from jax.experimental import pallas as pl
from jax.experimental.pallas import tpu as pltpu
from jax.experimental.pallas import tpu_sc as plsc

VOCAB = 100000
NEMBED = 128
BATCH = 4096
CTX = 20


_NC = 2
_NS = 16
_NW = _NC * _NS
_BPW = BATCH // _NW
_CHUNK = 4
_NCHUNK = _BPW // _CHUNK
_LANES = NEMBED // 16

_sc_mesh = plsc.VectorSubcoreMesh(core_axis_name="c", subcore_axis_name="s")


@functools.partial(
    pl.kernel,
    mesh=_sc_mesh,
    out_type=jax.ShapeDtypeStruct((BATCH, NEMBED), jnp.float32),
    scratch_types=[
        pltpu.VMEM((_BPW * CTX,), jnp.int32),
        pltpu.VMEM((_CHUNK * CTX, NEMBED), jnp.float32),
        pltpu.VMEM((_CHUNK * CTX, NEMBED), jnp.float32),
        pltpu.VMEM((_BPW, NEMBED), jnp.float32),
        pltpu.SemaphoreType.DMA,
        pltpu.SemaphoreType.DMA,
    ],
)
def _gather_sum(idx_hbm, table_hbm, out_hbm, idx_v, rows_a, rows_b, acc_v,
                sem_a, sem_b):
    wid = lax.axis_index("s") * _NC + lax.axis_index("c")
    base = wid * _BPW
    n_idx = _CHUNK * CTX
    pltpu.sync_copy(idx_hbm.at[pl.ds(base * CTX, _BPW * CTX)], idx_v)

    def _start(ci, buf, sem):
        pltpu.async_copy(table_hbm.at[idx_v.at[pl.ds(ci * n_idx, n_idx)]],
                         buf, sem)

    def _wait(buf, sem):
        pltpu.make_async_copy(table_hbm.at[pl.ds(0, n_idx)], buf, sem).wait()

    def _accum(ci, buf):
        for r in range(_CHUNK):
            accs = [buf[r * CTX, pl.ds(d * 16, 16)] for d in range(_LANES)]
            for c in range(1, CTX):
                for d in range(_LANES):
                    accs[d] = accs[d] + buf[r * CTX + c, pl.ds(d * 16, 16)]
            for d in range(_LANES):
                acc_v[ci * _CHUNK + r, pl.ds(d * 16, 16)] = accs[d]

    _start(0, rows_a, sem_a)
    _start(1, rows_b, sem_b)

    def pair_body(j, carry):
        c0 = j * 2
        _wait(rows_a, sem_a)
        _accum(c0, rows_a)

        @pl.when(c0 + 2 < _NCHUNK)
        def _():
            _start(c0 + 2, rows_a, sem_a)

        _wait(rows_b, sem_b)
        _accum(c0 + 1, rows_b)

        @pl.when(c0 + 3 < _NCHUNK)
        def _():
            _start(c0 + 3, rows_b, sem_b)

        return carry

    lax.fori_loop(0, _NCHUNK // 2, pair_body, 0)
    pltpu.sync_copy(acc_v, out_hbm.at[pl.ds(base, _BPW)])


_VB = 1536


def _mm_body(w_ref, x_ref, b_ref, o_ref):
    w = w_ref[...].astype(jnp.bfloat16)
    x = (x_ref[...] * (1.0 / CTX)).astype(jnp.bfloat16)
    acc = lax.dot_general(
        w, x, (((1,), (1,)), ((), ())), preferred_element_type=jnp.float32
    )
    o_ref[...] = acc + b_ref[...]


_matmul_t = pl.pallas_call(
    _mm_body,
    grid=(pl.cdiv(VOCAB, _VB),),
    in_specs=[
        pl.BlockSpec((_VB, NEMBED), lambda v: (v, 0)),
        pl.BlockSpec((BATCH, NEMBED), lambda v: (0, 0)),
        pl.BlockSpec((_VB, 1), lambda v: (v, 0)),
    ],
    out_specs=pl.BlockSpec((_VB, BATCH), lambda v: (v, 0)),
    out_shape=jax.ShapeDtypeStruct((VOCAB, BATCH), jnp.float32),
    compiler_params=pltpu.CompilerParams(
        dimension_semantics=("arbitrary",),
    ),
)


def kernel(inp, embed_table, W, b):
    idx = inp.reshape(-1).astype(jnp.int32)
    avg = _gather_sum(idx, embed_table)
    return _matmul_t(W, avg, b.reshape(VOCAB, 1)).T
